# Initial kernel scaffold; baseline (speedup 1.0000x reference)
import jax
import jax.numpy as jnp
from jax.experimental import pallas as pl


def kernel(index, f_x_i, y, phi_x_i, flag, epoch, u, prev_phi_x_i):
    raise NotImplementedError("write your pallas kernel here")


# speedup vs baseline: 4.8611x; 4.8611x over previous
import functools
import math

import jax
import jax.numpy as jnp
from jax import lax
from jax.experimental import pallas as pl
from jax.experimental.pallas import tpu as pltpu
from jax.experimental.pallas import tpu_sc as plsc

_TOTAL_EPOCHS = 150
_EPOCH_CONST = 10


def _make_sc_gather(n, b):
    info = plsc.get_sparse_core_info()
    nc, ns = info.num_cores, info.num_subcores
    nw = nc * ns
    assert b % (8 * nw) == 0
    bpw = b // nw
    mesh = plsc.VectorSubcoreMesh(core_axis_name="c", subcore_axis_name="s")

    @functools.partial(
        pl.kernel,
        mesh=mesh,
        out_type=jax.ShapeDtypeStruct((b,), jnp.float32),
        scratch_types=[
            pltpu.VMEM((bpw,), jnp.int32),
            pltpu.VMEM((bpw,), jnp.float32),
            pltpu.SemaphoreType.DMA,
        ],
    )
    def gather_k(u_hbm, idx_hbm, out_hbm, idx_v, vals_v, sem):
        wid = lax.axis_index("s") * nc + lax.axis_index("c")
        base = wid * bpw
        pltpu.sync_copy(idx_hbm.at[pl.ds(base, bpw)], idx_v)
        pltpu.async_copy(u_hbm.at[idx_v], vals_v, sem).wait()
        pltpu.sync_copy(vals_v, out_hbm.at[pl.ds(base, bpw)])

    return gather_k


def _phi_body(bin_size, c_dim, d_dim, cs, n_excl, n_sel,
              ut_ref, prev_ref, out_ref, w_ref):
    i = pl.program_id(0)
    steps = bin_size // cs

    @pl.when(i == 0)
    def _init():
        uv = ut_ref[...]
        jidx = lax.broadcasted_iota(jnp.int32, (bin_size, c_dim), 0)
        excl = jnp.zeros((bin_size, c_dim), jnp.bool_)
        for _ in range(n_excl):
            mvals = jnp.where(excl, -jnp.inf, uv)
            m = jnp.max(mvals, axis=0, keepdims=True)
            is_m = mvals == m
            cand = jnp.where(is_m, jidx, -1)
            jsel = jnp.max(cand, axis=0, keepdims=True)
            excl = excl | (jidx == jsel)
        w2 = jnp.where(excl, 0.0, 1.0 / n_sel).astype(jnp.float32)
        w_ref[...] = w2[:, :, None]
        out_ref[...] = jnp.zeros((c_dim, d_dim), jnp.float32)

    pv = prev_ref[...]
    acc = out_ref[...]
    for j in range(cs):
        wcol = jnp.squeeze(w_ref[pl.ds(i * cs + j, 1), :, :], axis=0)
        acc = acc + wcol * pv[j]
    out_ref[...] = acc

    @pl.when(i == steps - 1)
    def _fin():
        a = out_ref[...]
        nrm = jnp.sqrt(jnp.sum(a * a, axis=1, keepdims=True))
        out_ref[...] = a / nrm


def _make_phi_pass(bin_size, c_dim, d_dim, n_excl, n_sel):
    cs = 20
    steps = bin_size // cs
    body = functools.partial(_phi_body, bin_size, c_dim, d_dim, cs,
                             n_excl, n_sel)
    return pl.pallas_call(
        body,
        grid=(steps,),
        in_specs=[
            pl.BlockSpec((bin_size, c_dim), lambda i: (0, 0)),
            pl.BlockSpec((cs, c_dim, d_dim), lambda i: (i, 0, 0)),
        ],
        out_specs=pl.BlockSpec((c_dim, d_dim), lambda i: (0, 0)),
        out_shape=jax.ShapeDtypeStruct((c_dim, d_dim), jnp.float32),
        scratch_shapes=[pltpu.VMEM((bin_size, c_dim, 1), jnp.float32)],
    )


def _loss_body(b_dim, c_dim, d_dim, bs, eps,
               f_ref, y_ref, phi_ref, ug_ref, hcb_ref, out_ref):
    i = pl.program_id(0)
    steps = b_dim // bs

    f = f_ref[...]
    yv = y_ref[...]
    phi = phi_ref[...]
    ug = ug_ref[...]
    hcb = hcb_ref[...]

    fm = jnp.max(f, axis=1, keepdims=True)
    ef = jnp.exp(f - fm)
    f_sm = ef / jnp.sum(ef, axis=1, keepdims=True)

    nrm = jnp.sqrt(jnp.sum(phi * phi, axis=1, keepdims=True))
    h_i = phi / nrm
    dots = lax.dot_general(h_i, hcb, (((1,), (1,)), ((), ())),
                           preferred_element_type=jnp.float32)
    y_bar = dots * yv
    y_bar = jnp.where(y_bar > 0.0, y_bar, 0.0)

    u_by = ug * yv
    t = jnp.clip(f_sm + u_by, eps, 1.0)
    l1 = -jnp.sum(y_bar * jnp.log(t))

    cidx = lax.broadcasted_iota(jnp.int32, (bs, c_dim), 1)
    is_max = f == fm
    first = jnp.min(jnp.where(is_max, cidx, c_dim), axis=1, keepdims=True)
    y_hat = (cidx == first).astype(jnp.float32)
    d2 = y_hat + u_by - yv
    l2 = jnp.sum(d2 * d2)

    @pl.when(i == 0)
    def _init():
        out_ref[...] = jnp.zeros((1, 1), jnp.float32)

    out_ref[...] = out_ref[...] + (l1 + l2)

    @pl.when(i == steps - 1)
    def _fin():
        out_ref[...] = out_ref[...] * (1.0 / b_dim)


def _make_loss_pass(b_dim, c_dim, d_dim):
    bs = 512
    steps = b_dim // bs
    body = functools.partial(_loss_body, b_dim, c_dim, d_dim, bs, 1e-4)
    return pl.pallas_call(
        body,
        grid=(steps,),
        in_specs=[
            pl.BlockSpec((bs, c_dim), lambda i: (i, 0)),
            pl.BlockSpec((bs, c_dim), lambda i: (i, 0)),
            pl.BlockSpec((bs, d_dim), lambda i: (i, 0)),
            pl.BlockSpec((bs, 1), lambda i: (i, 0)),
            pl.BlockSpec((c_dim, d_dim), lambda i: (0, 0)),
        ],
        out_specs=pl.BlockSpec((1, 1), lambda i: (0, 0)),
        out_shape=jax.ShapeDtypeStruct((1, 1), jnp.float32),
    )


def kernel(index, f_x_i, y, phi_x_i, flag, epoch, u, prev_phi_x_i):
    b_dim, c_dim = f_x_i.shape
    d_dim = phi_x_i.shape[1]
    n = u.shape[0]
    bin_size = n // c_dim
    percent = math.ceil(50 - 50 / _TOTAL_EPOCHS * _EPOCH_CONST + 50)
    n_sel = int(bin_size / 100 * percent)
    n_excl = bin_size - n_sel

    u_flat = u.reshape(n)
    ut = u.reshape(bin_size, c_dim)
    prev3 = prev_phi_x_i.reshape(bin_size, c_dim, d_dim)

    ug = _make_sc_gather(n, b_dim)(u_flat, index.astype(jnp.int32))
    hcb = _make_phi_pass(bin_size, c_dim, d_dim, n_excl, n_sel)(ut, prev3)
    loss = _make_loss_pass(b_dim, c_dim, d_dim)(
        f_x_i, y, phi_x_i, ug.reshape(b_dim, 1), hcb)
    return loss[0, 0] + 0.0 * jnp.asarray(epoch, dtype=jnp.float32)


# speedup vs baseline: 9.1828x; 1.8890x over previous
import functools
import math

import jax
import jax.numpy as jnp
from jax import lax
from jax.experimental import pallas as pl
from jax.experimental.pallas import tpu as pltpu
from jax.experimental.pallas import tpu_sc as plsc

_TOTAL_EPOCHS = 150
_EPOCH_CONST = 10


def _make_sc_gather(n, b):
    info = plsc.get_sparse_core_info()
    nc, ns = info.num_cores, info.num_subcores
    nw = nc * ns
    assert b % (8 * nw) == 0
    bpw = b // nw
    mesh = plsc.VectorSubcoreMesh(core_axis_name="c", subcore_axis_name="s")

    @functools.partial(
        pl.kernel,
        mesh=mesh,
        out_type=jax.ShapeDtypeStruct((b,), jnp.float32),
        scratch_types=[
            pltpu.VMEM((bpw,), jnp.int32),
            pltpu.VMEM((bpw,), jnp.float32),
            pltpu.SemaphoreType.DMA,
        ],
    )
    def gather_k(u_hbm, idx_hbm, out_hbm, idx_v, vals_v, sem):
        wid = lax.axis_index("s") * nc + lax.axis_index("c")
        base = wid * bpw
        pltpu.sync_copy(idx_hbm.at[pl.ds(base, bpw)], idx_v)
        pltpu.async_copy(u_hbm.at[idx_v], vals_v, sem).wait()
        pltpu.sync_copy(vals_v, out_hbm.at[pl.ds(base, bpw)])

    return gather_k


def _w_body(bin_size, c_dim, n_excl, n_sel, ut_ref, w_ref):
    uv = ut_ref[...]
    jidx = lax.broadcasted_iota(jnp.int32, (bin_size, c_dim), 0)
    excl = jnp.zeros((bin_size, c_dim), jnp.bool_)
    for _ in range(n_excl):
        mvals = jnp.where(excl, -jnp.inf, uv)
        m = jnp.max(mvals, axis=0, keepdims=True)
        is_m = mvals == m
        cand = jnp.where(is_m, jidx, -1)
        jsel = jnp.max(cand, axis=0, keepdims=True)
        excl = excl | (jidx == jsel)
    w_ref[...] = jnp.where(excl, 0.0, 1.0 / n_sel).astype(jnp.float32)


def _make_w_pass(bin_size, c_dim, n_excl, n_sel):
    body = functools.partial(_w_body, bin_size, c_dim, n_excl, n_sel)
    return pl.pallas_call(
        body,
        out_shape=jax.ShapeDtypeStruct((bin_size, c_dim), jnp.float32),
    )


def _phi_body(n, c_dim, d_dim, rs, w_ref, prev_ref, out_ref, oh_ref):
    i = pl.program_id(0)
    steps = n // rs

    @pl.when(i == 0)
    def _init():
        q = lax.broadcasted_iota(jnp.int32, (c_dim, rs), 1)
        cc = lax.broadcasted_iota(jnp.int32, (c_dim, rs), 0)
        qmod = q - (q // c_dim) * c_dim
        oh_ref[...] = (qmod == cc).astype(jnp.float32)
        out_ref[...] = jnp.zeros((c_dim, d_dim), jnp.float32)

    wrow = w_ref[...].reshape(1, rs)
    sw = oh_ref[...] * wrow
    out_ref[...] += lax.dot_general(
        sw, prev_ref[...], (((1,), (0,)), ((), ())),
        preferred_element_type=jnp.float32)

    @pl.when(i == steps - 1)
    def _fin():
        a = out_ref[...]
        nrm = jnp.sqrt(jnp.sum(a * a, axis=1, keepdims=True))
        out_ref[...] = a / nrm


def _make_phi_pass(n, c_dim, d_dim):
    rs = 2000
    steps = n // rs
    body = functools.partial(_phi_body, n, c_dim, d_dim, rs)
    return pl.pallas_call(
        body,
        grid=(steps,),
        in_specs=[
            pl.BlockSpec((1, 1, rs), lambda i: (i, 0, 0)),
            pl.BlockSpec((rs, d_dim), lambda i: (i, 0)),
        ],
        out_specs=pl.BlockSpec((c_dim, d_dim), lambda i: (0, 0)),
        out_shape=jax.ShapeDtypeStruct((c_dim, d_dim), jnp.float32),
        scratch_shapes=[pltpu.VMEM((c_dim, rs), jnp.float32)],
    )


def _loss_body(b_dim, c_dim, d_dim, bs, eps,
               f_ref, y_ref, phi_ref, ug_ref, hcb_ref, out_ref, acc_ref):
    i = pl.program_id(0)
    steps = b_dim // bs

    f = f_ref[...]
    yv = y_ref[...]
    phi = phi_ref[...]
    ug = ug_ref[...]
    hcb = hcb_ref[...]

    fm = jnp.max(f, axis=1, keepdims=True)
    ef = jnp.exp(f - fm)
    f_sm = ef / jnp.sum(ef, axis=1, keepdims=True)

    nrm = jnp.sqrt(jnp.sum(phi * phi, axis=1, keepdims=True))
    h_i = phi / nrm
    dots = lax.dot_general(h_i, hcb, (((1,), (1,)), ((), ())),
                           preferred_element_type=jnp.float32)
    y_bar = dots * yv
    y_bar = jnp.where(y_bar > 0.0, y_bar, 0.0)

    u_by = ug * yv
    t = jnp.clip(f_sm + u_by, eps, 1.0)

    cidx = lax.broadcasted_iota(jnp.int32, (bs, c_dim), 1)
    is_max = f == fm
    first = jnp.min(jnp.where(is_max, cidx, c_dim), axis=1, keepdims=True)
    y_hat = (cidx == first).astype(jnp.float32)
    d2 = y_hat + u_by - yv

    elts = d2 * d2 - y_bar * jnp.log(t)

    @pl.when(i == 0)
    def _init():
        acc_ref[...] = jnp.zeros((bs, c_dim), jnp.float32)

    acc_ref[...] += elts

    @pl.when(i == steps - 1)
    def _fin():
        out_ref[...] = jnp.sum(acc_ref[...]) * (1.0 / b_dim) + jnp.zeros(
            (1, 1), jnp.float32)


def _make_loss_pass(b_dim, c_dim, d_dim):
    bs = 1024
    steps = b_dim // bs
    body = functools.partial(_loss_body, b_dim, c_dim, d_dim, bs, 1e-4)
    return pl.pallas_call(
        body,
        grid=(steps,),
        in_specs=[
            pl.BlockSpec((bs, c_dim), lambda i: (i, 0)),
            pl.BlockSpec((bs, c_dim), lambda i: (i, 0)),
            pl.BlockSpec((bs, d_dim), lambda i: (i, 0)),
            pl.BlockSpec((bs, 1), lambda i: (i, 0)),
            pl.BlockSpec((c_dim, d_dim), lambda i: (0, 0)),
        ],
        out_specs=pl.BlockSpec((1, 1), lambda i: (0, 0)),
        out_shape=jax.ShapeDtypeStruct((1, 1), jnp.float32),
        scratch_shapes=[pltpu.VMEM((bs, c_dim), jnp.float32)],
    )


def kernel(index, f_x_i, y, phi_x_i, flag, epoch, u, prev_phi_x_i):
    b_dim, c_dim = f_x_i.shape
    d_dim = phi_x_i.shape[1]
    n = u.shape[0]
    bin_size = n // c_dim
    percent = math.ceil(50 - 50 / _TOTAL_EPOCHS * _EPOCH_CONST + 50)
    n_sel = int(bin_size / 100 * percent)
    n_excl = bin_size - n_sel

    rs = 2000
    ut = u.reshape(bin_size, c_dim)

    ug = _make_sc_gather(n, b_dim)(u.reshape(n), index.astype(jnp.int32))
    w = _make_w_pass(bin_size, c_dim, n_excl, n_sel)(ut)
    wflat = w.reshape(n // rs, 1, rs)
    hcb = _make_phi_pass(n, c_dim, d_dim)(wflat, prev_phi_x_i)
    loss = _make_loss_pass(b_dim, c_dim, d_dim)(
        f_x_i, y, phi_x_i, ug.reshape(b_dim, 1), hcb)
    return loss[0, 0] + 0.0 * jnp.asarray(epoch, dtype=jnp.float32)


# speedup vs baseline: 9.7068x; 1.0571x over previous
import functools
import math

import jax
import jax.numpy as jnp
from jax import lax
from jax.experimental import pallas as pl
from jax.experimental.pallas import tpu as pltpu
from jax.experimental.pallas import tpu_sc as plsc

_TOTAL_EPOCHS = 150
_EPOCH_CONST = 10


def _make_sc_gather(n, b):
    info = plsc.get_sparse_core_info()
    nc, ns = info.num_cores, info.num_subcores
    nw = nc * ns
    assert b % (8 * nw) == 0
    bpw = b // nw
    mesh = plsc.VectorSubcoreMesh(core_axis_name="c", subcore_axis_name="s")

    @functools.partial(
        pl.kernel,
        mesh=mesh,
        out_type=jax.ShapeDtypeStruct((b,), jnp.float32),
        scratch_types=[
            pltpu.VMEM((bpw,), jnp.int32),
            pltpu.VMEM((bpw,), jnp.float32),
            pltpu.SemaphoreType.DMA,
        ],
    )
    def gather_k(u_hbm, idx_hbm, out_hbm, idx_v, vals_v, sem):
        wid = lax.axis_index("s") * nc + lax.axis_index("c")
        base = wid * bpw
        pltpu.sync_copy(idx_hbm.at[pl.ds(base, bpw)], idx_v)
        pltpu.async_copy(u_hbm.at[idx_v], vals_v, sem).wait()
        pltpu.sync_copy(vals_v, out_hbm.at[pl.ds(base, bpw)])

    return gather_k


def _w_body(bin_size, c_dim, n_excl, n_sel, ut_ref, w_ref):
    uv = ut_ref[...]
    jidx = lax.broadcasted_iota(jnp.int32, (bin_size, c_dim), 0)
    excl = jnp.zeros((bin_size, c_dim), jnp.bool_)
    for _ in range(n_excl):
        mvals = jnp.where(excl, -jnp.inf, uv)
        m = jnp.max(mvals, axis=0, keepdims=True)
        is_m = mvals == m
        cand = jnp.where(is_m, jidx, -1)
        jsel = jnp.max(cand, axis=0, keepdims=True)
        excl = excl | (jidx == jsel)
    w_ref[...] = jnp.where(excl, 0.0, 1.0 / n_sel).astype(jnp.float32)


def _make_w_pass(bin_size, c_dim, n_excl, n_sel):
    body = functools.partial(_w_body, bin_size, c_dim, n_excl, n_sel)
    return pl.pallas_call(
        body,
        out_shape=jax.ShapeDtypeStruct((bin_size, c_dim), jnp.float32),
    )


def _phi_body(n, c_dim, d_dim, rs, w_ref, prev_ref, out_ref, oh_ref):
    i = pl.program_id(0)
    steps = n // rs

    @pl.when(i == 0)
    def _init():
        q = lax.broadcasted_iota(jnp.int32, (c_dim, rs), 1)
        cc = lax.broadcasted_iota(jnp.int32, (c_dim, rs), 0)
        qmod = q - (q // c_dim) * c_dim
        oh_ref[...] = (qmod == cc).astype(jnp.float32)
        out_ref[...] = jnp.zeros((c_dim, d_dim), jnp.float32)

    wrow = w_ref[...].reshape(1, rs)
    sw = oh_ref[...] * wrow
    out_ref[...] += lax.dot_general(
        sw, prev_ref[...], (((1,), (0,)), ((), ())),
        preferred_element_type=jnp.float32)

    @pl.when(i == steps - 1)
    def _fin():
        a = out_ref[...]
        nrm = jnp.sqrt(jnp.sum(a * a, axis=1, keepdims=True))
        out_ref[...] = a / nrm


def _make_phi_pass(n, c_dim, d_dim):
    rs = 5000
    steps = n // rs
    body = functools.partial(_phi_body, n, c_dim, d_dim, rs)
    return pl.pallas_call(
        body,
        grid=(steps,),
        in_specs=[
            pl.BlockSpec((1, 1, rs), lambda i: (i, 0, 0)),
            pl.BlockSpec((rs, d_dim), lambda i: (i, 0)),
        ],
        out_specs=pl.BlockSpec((c_dim, d_dim), lambda i: (0, 0)),
        out_shape=jax.ShapeDtypeStruct((c_dim, d_dim), jnp.float32),
        scratch_shapes=[pltpu.VMEM((c_dim, rs), jnp.float32)],
    )


def _loss_body(b_dim, c_dim, d_dim, bs, eps,
               f_ref, y_ref, phi_ref, ug_ref, hcb_ref, out_ref, acc_ref):
    i = pl.program_id(0)
    steps = b_dim // bs

    f = f_ref[...]
    yv = y_ref[...]
    phi = phi_ref[...]
    ug = ug_ref[...]
    hcb = hcb_ref[...]

    fm = jnp.max(f, axis=1, keepdims=True)
    ef = jnp.exp(f - fm)
    f_sm = ef / jnp.sum(ef, axis=1, keepdims=True)

    nrm = jnp.sqrt(jnp.sum(phi * phi, axis=1, keepdims=True))
    h_i = phi / nrm
    dots = lax.dot_general(h_i, hcb, (((1,), (1,)), ((), ())),
                           preferred_element_type=jnp.float32)
    y_bar = dots * yv
    y_bar = jnp.where(y_bar > 0.0, y_bar, 0.0)

    u_by = ug * yv
    t = jnp.clip(f_sm + u_by, eps, 1.0)

    cidx = lax.broadcasted_iota(jnp.int32, (bs, c_dim), 1)
    is_max = f == fm
    first = jnp.min(jnp.where(is_max, cidx, c_dim), axis=1, keepdims=True)
    y_hat = (cidx == first).astype(jnp.float32)
    d2 = y_hat + u_by - yv

    elts = d2 * d2 - y_bar * jnp.log(t)

    @pl.when(i == 0)
    def _init():
        acc_ref[...] = jnp.zeros((bs, c_dim), jnp.float32)

    acc_ref[...] += elts

    @pl.when(i == steps - 1)
    def _fin():
        out_ref[...] = jnp.sum(acc_ref[...]) * (1.0 / b_dim) + jnp.zeros(
            (1, 1), jnp.float32)


def _make_loss_pass(b_dim, c_dim, d_dim):
    bs = 2048
    steps = b_dim // bs
    body = functools.partial(_loss_body, b_dim, c_dim, d_dim, bs, 1e-4)
    return pl.pallas_call(
        body,
        grid=(steps,),
        in_specs=[
            pl.BlockSpec((bs, c_dim), lambda i: (i, 0)),
            pl.BlockSpec((bs, c_dim), lambda i: (i, 0)),
            pl.BlockSpec((bs, d_dim), lambda i: (i, 0)),
            pl.BlockSpec((bs, 1), lambda i: (i, 0)),
            pl.BlockSpec((c_dim, d_dim), lambda i: (0, 0)),
        ],
        out_specs=pl.BlockSpec((1, 1), lambda i: (0, 0)),
        out_shape=jax.ShapeDtypeStruct((1, 1), jnp.float32),
        scratch_shapes=[pltpu.VMEM((bs, c_dim), jnp.float32)],
    )


def kernel(index, f_x_i, y, phi_x_i, flag, epoch, u, prev_phi_x_i):
    b_dim, c_dim = f_x_i.shape
    d_dim = phi_x_i.shape[1]
    n = u.shape[0]
    bin_size = n // c_dim
    percent = math.ceil(50 - 50 / _TOTAL_EPOCHS * _EPOCH_CONST + 50)
    n_sel = int(bin_size / 100 * percent)
    n_excl = bin_size - n_sel

    rs = 5000
    u_flat = u.reshape(n)
    ut = u_flat.reshape(bin_size, c_dim)

    ug = _make_sc_gather(n, b_dim)(u_flat, index.astype(jnp.int32))
    w = _make_w_pass(bin_size, c_dim, n_excl, n_sel)(ut)
    wflat = w.reshape(n // rs, 1, rs)
    hcb = _make_phi_pass(n, c_dim, d_dim)(wflat, prev_phi_x_i)
    loss = _make_loss_pass(b_dim, c_dim, d_dim)(
        f_x_i, y, phi_x_i, ug.reshape(b_dim, 1), hcb)
    return loss[0, 0] + 0.0 * jnp.asarray(epoch, dtype=jnp.float32)


# speedup vs baseline: 9.9842x; 1.0286x over previous
import functools
import math

import jax
import jax.numpy as jnp
from jax import lax
from jax.experimental import pallas as pl
from jax.experimental.pallas import tpu as pltpu
from jax.experimental.pallas import tpu_sc as plsc

_TOTAL_EPOCHS = 150
_EPOCH_CONST = 10


def _make_sc_gather(n, b):
    info = plsc.get_sparse_core_info()
    nc, ns = info.num_cores, info.num_subcores
    nw = nc * ns
    assert b % (8 * nw) == 0 and n % 8 == 0
    bpw = b // nw
    mesh = plsc.VectorSubcoreMesh(core_axis_name="c", subcore_axis_name="s")

    @functools.partial(
        pl.kernel,
        mesh=mesh,
        out_type=jax.ShapeDtypeStruct((b,), jnp.int32),
        scratch_types=[
            pltpu.VMEM((bpw,), jnp.int32),
            pltpu.VMEM((bpw,), jnp.int32),
            pltpu.SemaphoreType.DMA,
        ],
    )
    def gather_k(tbl_hbm, out_hbm, idx_v, vals_v, sem):
        wid = lax.axis_index("s") * nc + lax.axis_index("c")
        base = wid * bpw
        pltpu.sync_copy(tbl_hbm.at[pl.ds(n + base, bpw)], idx_v)
        pltpu.async_copy(tbl_hbm.at[idx_v], vals_v, sem).wait()
        pltpu.sync_copy(vals_v, out_hbm.at[pl.ds(base, bpw)])

    return gather_k


def _w_body(bin_size, c_dim, n_excl, n_sel, ut_ref, w_ref):
    uv = ut_ref[...]
    jidx = lax.broadcasted_iota(jnp.int32, (bin_size, c_dim), 0)
    excl = jnp.zeros((bin_size, c_dim), jnp.bool_)
    for _ in range(n_excl):
        mvals = jnp.where(excl, -jnp.inf, uv)
        m = jnp.max(mvals, axis=0, keepdims=True)
        is_m = mvals == m
        cand = jnp.where(is_m, jidx, -1)
        jsel = jnp.max(cand, axis=0, keepdims=True)
        excl = excl | (jidx == jsel)
    w_ref[...] = jnp.where(excl, 0.0, 1.0 / n_sel).astype(jnp.float32)


def _make_w_pass(bin_size, c_dim, n_excl, n_sel):
    body = functools.partial(_w_body, bin_size, c_dim, n_excl, n_sel)
    return pl.pallas_call(
        body,
        out_shape=jax.ShapeDtypeStruct((bin_size, c_dim), jnp.float32),
    )


def _phi_body(n, c_dim, d_dim, rs, w_ref, prev_ref, out_ref, oh_ref):
    i = pl.program_id(0)
    steps = n // rs

    @pl.when(i == 0)
    def _init():
        q = lax.broadcasted_iota(jnp.int32, (c_dim, rs), 1)
        cc = lax.broadcasted_iota(jnp.int32, (c_dim, rs), 0)
        qmod = q - (q // c_dim) * c_dim
        oh_ref[...] = (qmod == cc).astype(jnp.float32)
        out_ref[...] = jnp.zeros((c_dim, d_dim), jnp.float32)

    wrow = w_ref[...].reshape(1, rs)
    sw = oh_ref[...] * wrow
    out_ref[...] += lax.dot_general(
        sw, prev_ref[...], (((1,), (0,)), ((), ())),
        preferred_element_type=jnp.float32)

    @pl.when(i == steps - 1)
    def _fin():
        a = out_ref[...]
        nrm = jnp.sqrt(jnp.sum(a * a, axis=1, keepdims=True))
        out_ref[...] = a / nrm


def _make_phi_pass(n, c_dim, d_dim):
    rs = 5000
    steps = n // rs
    body = functools.partial(_phi_body, n, c_dim, d_dim, rs)
    return pl.pallas_call(
        body,
        grid=(steps,),
        in_specs=[
            pl.BlockSpec((1, 1, rs), lambda i: (i, 0, 0)),
            pl.BlockSpec((rs, d_dim), lambda i: (i, 0)),
        ],
        out_specs=pl.BlockSpec((c_dim, d_dim), lambda i: (0, 0)),
        out_shape=jax.ShapeDtypeStruct((c_dim, d_dim), jnp.float32),
        scratch_shapes=[pltpu.VMEM((c_dim, rs), jnp.float32)],
    )


def _loss_body(b_dim, c_dim, d_dim, bs, eps,
               f_ref, y_ref, phi_ref, ug_ref, hcb_ref, out_ref, acc_ref):
    i = pl.program_id(0)
    steps = b_dim // bs

    f = f_ref[...]
    yv = y_ref[...]
    phi = phi_ref[...]
    ug = ug_ref[...].reshape(bs, 1)
    hcb = hcb_ref[...]

    fm = jnp.max(f, axis=1, keepdims=True)
    ef = jnp.exp(f - fm)
    f_sm = ef / jnp.sum(ef, axis=1, keepdims=True)

    nrm = jnp.sqrt(jnp.sum(phi * phi, axis=1, keepdims=True))
    h_i = phi / nrm
    dots = lax.dot_general(h_i, hcb, (((1,), (1,)), ((), ())),
                           preferred_element_type=jnp.float32)
    y_bar = dots * yv
    y_bar = jnp.where(y_bar > 0.0, y_bar, 0.0)

    u_by = ug * yv
    t = jnp.clip(f_sm + u_by, eps, 1.0)

    cidx = lax.broadcasted_iota(jnp.int32, (bs, c_dim), 1)
    is_max = f == fm
    first = jnp.min(jnp.where(is_max, cidx, c_dim), axis=1, keepdims=True)
    y_hat = (cidx == first).astype(jnp.float32)
    d2 = y_hat + u_by - yv

    elts = d2 * d2 - y_bar * jnp.log(t)

    @pl.when(i == 0)
    def _init():
        acc_ref[...] = jnp.zeros((bs, c_dim), jnp.float32)

    acc_ref[...] += elts

    @pl.when(i == steps - 1)
    def _fin():
        out_ref[...] = jnp.sum(acc_ref[...]) * (1.0 / b_dim) + jnp.zeros(
            (1, 1), jnp.float32)


def _make_loss_pass(b_dim, c_dim, d_dim):
    bs = 2048
    steps = b_dim // bs
    body = functools.partial(_loss_body, b_dim, c_dim, d_dim, bs, 1e-4)
    return pl.pallas_call(
        body,
        grid=(steps,),
        in_specs=[
            pl.BlockSpec((bs, c_dim), lambda i: (i, 0)),
            pl.BlockSpec((bs, c_dim), lambda i: (i, 0)),
            pl.BlockSpec((bs, d_dim), lambda i: (i, 0)),
            pl.BlockSpec((bs,), lambda i: (i,)),
            pl.BlockSpec((c_dim, d_dim), lambda i: (0, 0)),
        ],
        out_specs=pl.BlockSpec((1, 1), lambda i: (0, 0)),
        out_shape=jax.ShapeDtypeStruct((1, 1), jnp.float32),
        scratch_shapes=[pltpu.VMEM((bs, c_dim), jnp.float32)],
    )


def kernel(index, f_x_i, y, phi_x_i, flag, epoch, u, prev_phi_x_i):
    b_dim, c_dim = f_x_i.shape
    d_dim = phi_x_i.shape[1]
    n = u.shape[0]
    bin_size = n // c_dim
    percent = math.ceil(50 - 50 / _TOTAL_EPOCHS * _EPOCH_CONST + 50)
    n_sel = int(bin_size / 100 * percent)
    n_excl = bin_size - n_sel

    rs = 5000
    u_flat = u.reshape(n)
    ut = u_flat.reshape(bin_size, c_dim)

    tbl_idx = jnp.concatenate(
        [lax.bitcast_convert_type(u_flat, jnp.int32),
         index.astype(jnp.int32)])
    ug = lax.bitcast_convert_type(
        _make_sc_gather(n, b_dim)(tbl_idx), jnp.float32)
    w = _make_w_pass(bin_size, c_dim, n_excl, n_sel)(ut)
    wflat = w.reshape(n // rs, 1, rs)
    hcb = _make_phi_pass(n, c_dim, d_dim)(wflat, prev_phi_x_i)
    loss = _make_loss_pass(b_dim, c_dim, d_dim)(f_x_i, y, phi_x_i, ug, hcb)
    return loss[0, 0] + 0.0 * jnp.asarray(epoch, dtype=jnp.float32)


# speedup vs baseline: 12.7151x; 1.2735x over previous
import functools
import math

import jax
import jax.numpy as jnp
from jax import lax
from jax.experimental import pallas as pl
from jax.experimental.pallas import tpu as pltpu
from jax.experimental.pallas import tpu_sc as plsc

_TOTAL_EPOCHS = 150
_EPOCH_CONST = 10


def _make_sc_gather(n, b):
    info = plsc.get_sparse_core_info()
    nc, ns = info.num_cores, info.num_subcores
    nw = nc * ns
    assert b % (8 * nw) == 0 and n % 8 == 0
    bpw = b // nw
    mesh = plsc.VectorSubcoreMesh(core_axis_name="c", subcore_axis_name="s")

    @functools.partial(
        pl.kernel,
        mesh=mesh,
        out_type=jax.ShapeDtypeStruct((b,), jnp.int32),
        scratch_types=[
            pltpu.VMEM((bpw,), jnp.int32),
            pltpu.VMEM((bpw,), jnp.int32),
            pltpu.SemaphoreType.DMA,
        ],
    )
    def gather_k(tbl_hbm, out_hbm, idx_v, vals_v, sem):
        wid = lax.axis_index("s") * nc + lax.axis_index("c")
        base = wid * bpw
        pltpu.sync_copy(tbl_hbm.at[pl.ds(n + base, bpw)], idx_v)
        pltpu.async_copy(tbl_hbm.at[idx_v], vals_v, sem).wait()
        pltpu.sync_copy(vals_v, out_hbm.at[pl.ds(base, bpw)])

    return gather_k


def _w_body(bin_size, c_dim, n_excl, n_sel, ut_ref, w_ref):
    uv = ut_ref[...]
    jidx = lax.broadcasted_iota(jnp.int32, (bin_size, c_dim), 0)
    excl = jnp.zeros((bin_size, c_dim), jnp.bool_)
    for _ in range(n_excl):
        mvals = jnp.where(excl, -jnp.inf, uv)
        m = jnp.max(mvals, axis=0, keepdims=True)
        is_m = mvals == m
        cand = jnp.where(is_m, jidx, -1)
        jsel = jnp.max(cand, axis=0, keepdims=True)
        excl = excl | (jidx == jsel)
    w_ref[...] = jnp.where(excl, 0.0, 1.0 / n_sel).astype(jnp.float32)


def _make_w_pass(bin_size, c_dim, n_excl, n_sel):
    body = functools.partial(_w_body, bin_size, c_dim, n_excl, n_sel)
    return pl.pallas_call(
        body,
        out_shape=jax.ShapeDtypeStruct((bin_size, c_dim), jnp.float32),
    )


def _phi_body(n, c_dim, d_dim, rs, w_ref, prev_ref, out_ref, oh_ref):
    i = pl.program_id(0)
    steps = n // rs

    @pl.when(i == 0)
    def _init():
        q = lax.broadcasted_iota(jnp.int32, (c_dim, rs), 1)
        cc = lax.broadcasted_iota(jnp.int32, (c_dim, rs), 0)
        qmod = q - (q // c_dim) * c_dim
        oh_ref[...] = (qmod == cc).astype(jnp.float32)
        out_ref[...] = jnp.zeros((c_dim, d_dim), jnp.float32)

    wrow = w_ref[...].reshape(1, rs)
    sw = oh_ref[...] * wrow
    out_ref[...] += lax.dot_general(
        sw, prev_ref[...], (((1,), (0,)), ((), ())),
        preferred_element_type=jnp.float32)

    @pl.when(i == steps - 1)
    def _fin():
        a = out_ref[...]
        nrm = jnp.sqrt(jnp.sum(a * a, axis=1, keepdims=True))
        out_ref[...] = a / nrm


def _make_phi_pass(n, c_dim, d_dim):
    rs = 5000
    steps = n // rs
    body = functools.partial(_phi_body, n, c_dim, d_dim, rs)
    return pl.pallas_call(
        body,
        grid=(steps,),
        in_specs=[
            pl.BlockSpec((1, 1, rs), lambda i: (i, 0, 0)),
            pl.BlockSpec((rs, d_dim), lambda i: (i, 0)),
        ],
        out_specs=pl.BlockSpec((c_dim, d_dim), lambda i: (0, 0)),
        out_shape=jax.ShapeDtypeStruct((c_dim, d_dim), jnp.float32),
        scratch_shapes=[pltpu.VMEM((c_dim, rs), jnp.float32)],
    )


def _loss_body(b_dim, c_dim, d_dim, bs, eps,
               ft_ref, yt_ref, phi_ref, ug_ref, hcb_ref, out_ref, acc_ref):
    i = pl.program_id(0)
    steps = b_dim // bs

    ft = ft_ref[...]
    yt = yt_ref[...]
    phi = phi_ref[...]
    ug = ug_ref[...].reshape(1, bs)
    hcb = hcb_ref[...]

    fm = jnp.max(ft, axis=0, keepdims=True)
    ef = jnp.exp(ft - fm)
    f_sm = ef / jnp.sum(ef, axis=0, keepdims=True)

    ss = lax.dot_general(jnp.ones((1, d_dim), jnp.float32), phi * phi,
                         (((1,), (1,)), ((), ())),
                         preferred_element_type=jnp.float32)
    inv_nrm = 1.0 / jnp.sqrt(ss)
    dots = lax.dot_general(hcb, phi, (((1,), (1,)), ((), ())),
                           preferred_element_type=jnp.float32)
    y_bar = (dots * inv_nrm) * yt
    y_bar = jnp.where(y_bar > 0.0, y_bar, 0.0)

    u_by = ug * yt
    t = jnp.clip(f_sm + u_by, eps, 1.0)

    cidx = lax.broadcasted_iota(jnp.int32, (c_dim, bs), 0)
    is_max = ft == fm
    first = jnp.min(jnp.where(is_max, cidx, c_dim), axis=0, keepdims=True)
    y_hat = (cidx == first).astype(jnp.float32)
    d2 = y_hat + u_by - yt

    elts = d2 * d2 - y_bar * jnp.log(t)

    @pl.when(i == 0)
    def _init():
        acc_ref[...] = jnp.zeros((c_dim, bs), jnp.float32)

    acc_ref[...] += elts

    @pl.when(i == steps - 1)
    def _fin():
        out_ref[...] = jnp.sum(acc_ref[...]) * (1.0 / b_dim) + jnp.zeros(
            (1, 1), jnp.float32)


def _make_loss_pass(b_dim, c_dim, d_dim):
    bs = 2048
    steps = b_dim // bs
    body = functools.partial(_loss_body, b_dim, c_dim, d_dim, bs, 1e-4)
    return pl.pallas_call(
        body,
        grid=(steps,),
        in_specs=[
            pl.BlockSpec((c_dim, bs), lambda i: (0, i)),
            pl.BlockSpec((c_dim, bs), lambda i: (0, i)),
            pl.BlockSpec((bs, d_dim), lambda i: (i, 0)),
            pl.BlockSpec((bs,), lambda i: (i,)),
            pl.BlockSpec((c_dim, d_dim), lambda i: (0, 0)),
        ],
        out_specs=pl.BlockSpec((1, 1), lambda i: (0, 0)),
        out_shape=jax.ShapeDtypeStruct((1, 1), jnp.float32),
        scratch_shapes=[pltpu.VMEM((c_dim, bs), jnp.float32)],
    )


def kernel(index, f_x_i, y, phi_x_i, flag, epoch, u, prev_phi_x_i):
    b_dim, c_dim = f_x_i.shape
    d_dim = phi_x_i.shape[1]
    n = u.shape[0]
    bin_size = n // c_dim
    percent = math.ceil(50 - 50 / _TOTAL_EPOCHS * _EPOCH_CONST + 50)
    n_sel = int(bin_size / 100 * percent)
    n_excl = bin_size - n_sel

    rs = 5000
    u_flat = u.reshape(n)
    ut = u_flat.reshape(bin_size, c_dim)

    tbl_idx = jnp.concatenate(
        [lax.bitcast_convert_type(u_flat, jnp.int32),
         index.astype(jnp.int32)])
    ug = lax.bitcast_convert_type(
        _make_sc_gather(n, b_dim)(tbl_idx), jnp.float32)
    w = _make_w_pass(bin_size, c_dim, n_excl, n_sel)(ut)
    wflat = w.reshape(n // rs, 1, rs)
    hcb = _make_phi_pass(n, c_dim, d_dim)(wflat, prev_phi_x_i)
    loss = _make_loss_pass(b_dim, c_dim, d_dim)(
        f_x_i.T, y.T, phi_x_i, ug, hcb)
    return loss[0, 0] + 0.0 * jnp.asarray(epoch, dtype=jnp.float32)
